# baseline (speedup 1.0000x reference)
import dataclasses
import functools

import jax
import jax.numpy as jnp
from jax import lax
from jax.experimental import pallas as pl
from jax.experimental.pallas import tpu as pltpu
from jax.experimental.pallas import tpu_sc as plsc

_N = 10000
_E = 320000
_F = 128
_H = 128
_C = 16

_D1 = 80
_D2 = 16

_NCORES = 2
_NSUB = 16
_CH = _E // _NSUB
_B = 80
_NB = _CH // _B
_NCH = _N // _B
_DT = 4000
_SEG = _CH // _DT


def _dense1(features, w1, att_s, att_d):
  def body(x_ref, w_ref, s_ref, d_ref, tab_ref, as_ref, ad_ref, m_ref):
    xw = jnp.dot(x_ref[...], w_ref[...], preferred_element_type=jnp.float32)
    hh = _H // 2
    ones_col = (lax.broadcasted_iota(jnp.int32, (_N, _D1 - hh), 1) == 0)
    ones_col = ones_col.astype(jnp.float32)
    tab_ref[0, :, :hh] = xw[:, :hh]
    tab_ref[0, :, hh:] = ones_col
    tab_ref[1, :, :hh] = xw[:, hh:]
    tab_ref[1, :, hh:] = ones_col
    a_s = jnp.sum(xw * s_ref[...], axis=1)
    a_d = jnp.sum(xw * d_ref[...], axis=1)
    as_ref[...] = a_s[None, :]
    ad_ref[...] = a_d[None, :]
    m_ref[...] = jnp.concatenate(
        [jnp.max(a_s)[None, None], jnp.max(a_d)[None, None]], axis=1)

  return pl.pallas_call(
      body,
      out_shape=[
          jax.ShapeDtypeStruct((_NCORES, _N, _D1), jnp.float32),
          jax.ShapeDtypeStruct((1, _N), jnp.float32),
          jax.ShapeDtypeStruct((1, _N), jnp.float32),
          jax.ShapeDtypeStruct((1, 2), jnp.float32),
      ],
  )(features, w1, att_s, att_d)


def _mid(partial1, b1, w2, att_s, att_d):
  def body(p_ref, b_ref, w_ref, s_ref, d_ref, tab_ref, as_ref, ad_ref, m_ref):
    hh = _H // 2
    num = jnp.concatenate(
        [p_ref[0, :_N, :hh], p_ref[1, :_N, :hh]], axis=1)
    den = p_ref[0, :_N, hh:hh + 1] + 1e-16
    h = jax.nn.relu(num / den + b_ref[...])
    xw = jnp.dot(h, w_ref[...], preferred_element_type=jnp.float32)
    ch = _C // 2
    ones_col = (lax.broadcasted_iota(jnp.int32, (_N, _D2 - ch), 1) == 0)
    ones_col = ones_col.astype(jnp.float32)
    tab_ref[0, :, :ch] = xw[:, :ch]
    tab_ref[0, :, ch:] = ones_col
    tab_ref[1, :, :ch] = xw[:, ch:]
    tab_ref[1, :, ch:] = ones_col
    a_s = jnp.sum(xw * s_ref[...], axis=1)
    a_d = jnp.sum(xw * d_ref[...], axis=1)
    as_ref[...] = a_s[None, :]
    ad_ref[...] = a_d[None, :]
    m_ref[...] = jnp.concatenate(
        [jnp.max(a_s)[None, None], jnp.max(a_d)[None, None]], axis=1)

  return pl.pallas_call(
      body,
      out_shape=[
          jax.ShapeDtypeStruct((_NCORES, _N, _D2), jnp.float32),
          jax.ShapeDtypeStruct((1, _N), jnp.float32),
          jax.ShapeDtypeStruct((1, _N), jnp.float32),
          jax.ShapeDtypeStruct((1, 2), jnp.float32),
      ],
  )(partial1, b1, w2, att_s, att_d)


def _final(partial2, b2):
  def body(p_ref, b_ref, o_ref):
    ch = _C // 2
    num = jnp.concatenate(
        [p_ref[0, :_N, :ch], p_ref[1, :_N, :ch]], axis=1)
    den = p_ref[0, :_N, ch:ch + 1] + 1e-16
    o = num / den + b_ref[...]
    mx = jnp.max(o, axis=1, keepdims=True)
    shifted = o - mx
    lse = jnp.log(jnp.sum(jnp.exp(shifted), axis=1, keepdims=True))
    o_ref[...] = shifted - lse

  return pl.pallas_call(
      body,
      out_shape=jax.ShapeDtypeStruct((_N, _C), jnp.float32),
  )(partial2, b2)


def _sc_layer(tabs, src, dst, a_s, a_d, m16, d, nbuf):
  nd = d // 16
  assert _NB % nbuf == 0
  mesh = plsc.VectorSubcoreMesh(core_axis_name="c", subcore_axis_name="s")
  cp = pltpu.CompilerParams()
  if "needs_layout_passes" in pltpu.CompilerParams.__dataclass_fields__:
    cp = dataclasses.replace(cp, needs_layout_passes=False)
  if "use_tc_tiling_on_sc" in pltpu.CompilerParams.__dataclass_fields__:
    cp = dataclasses.replace(cp, use_tc_tiling_on_sc=False)

  @functools.partial(
      pl.kernel,
      mesh=mesh,
      compiler_params=cp,
      out_type=jax.ShapeDtypeStruct((_NCORES, _N, d), jnp.float32),
      scratch_types=[
          pltpu.VMEM((_N,), jnp.float32),
          pltpu.VMEM((_N,), jnp.float32),
          pltpu.VMEM((16,), jnp.float32),
          pltpu.VMEM((_CH,), jnp.int32),
          pltpu.VMEM((_NB, _B), jnp.int32),
          pltpu.VMEM((_DT,), jnp.int32),
          pltpu.VMEM((_B,), jnp.float32),
      ] + [pltpu.VMEM((_B, d), jnp.float32) for _ in range(nbuf)]
      + [pltpu.VMEM_SHARED((_N, d), jnp.float32)]
      + [pltpu.SemaphoreType.DMA for _ in range(2 * nbuf)],
  )
  def k(tab_hbm, src_hbm, dst_hbm, as_hbm, ad_hbm, m_hbm, out_hbm,
        as_v, ad_v, m_v, si_v, dr_v, dt_v, w_v, *rest):
    bufs = rest[:nbuf]
    acc_sh = rest[nbuf]
    gsems = rest[nbuf + 1:2 * nbuf + 1]
    ssems = rest[2 * nbuf + 1:]
    rows_a = bufs[0]
    c = lax.axis_index("c")
    s = lax.axis_index("s")

    pltpu.sync_copy(as_hbm, as_v)
    pltpu.sync_copy(ad_hbm, ad_v)
    pltpu.sync_copy(m_hbm, m_v)
    pltpu.sync_copy(src_hbm.at[pl.ds(s * _CH, _CH)], si_v)

    for seg in range(_SEG):
      pltpu.sync_copy(dst_hbm.at[pl.ds(s * _CH + seg * _DT, _DT)], dt_v)

      @pl.loop(0, _DT // _B)
      def _(b):
        for jj in range(_B // 16):
          dr_v[seg * (_DT // _B) + b, pl.ds(jj * 16, 16)] = (
              dt_v[pl.ds(b * _B + jj * 16, 16)])

    @pl.loop(0, _B)
    def _(r):
      for ch in range(nd):
        rows_a[r, pl.ds(ch * 16, 16)] = jnp.zeros((16,), jnp.float32)

    @pl.loop(0, (_NCH + _NSUB - 1) // _NSUB)
    def _(t):
      idx = t * _NSUB + s

      @pl.when(idx < _NCH)
      def _():
        pltpu.sync_copy(rows_a, acc_sh.at[pl.ds(idx * _B, _B)])

    plsc.subcore_barrier()

    mvec = m_v[...]

    def compute_w(blk):
      @plsc.parallel_loop(0, _B, 16, unroll=_B // 16)
      def _(j):
        sidx = si_v[pl.ds(blk * _B + j, 16)]
        didx = dr_v[blk, pl.ds(j, 16)]
        e = plsc.load_gather(as_v, [sidx]) + plsc.load_gather(ad_v, [didx])
        e = jnp.maximum(e, 0.2 * e)
        w_v[pl.ds(j, 16)] = jnp.exp(e - mvec)

    zc = jnp.zeros((16,), jnp.int32)

    def scale(rows_v):
      @plsc.parallel_loop(0, _B, unroll=16)
      def _(i):
        ws = plsc.load_gather(w_v, [zc + i])
        for ch in range(nd):
          sl = pl.ds(ch * 16, 16)
          rows_v[i, sl] = rows_v[i, sl] * ws

    @pl.loop(0, _NB, step=nbuf)
    def _(blk):
      @pl.when(blk > 0)
      def _():
        for q in range(nbuf):
          pltpu.make_async_copy(bufs[q], acc_sh.at[dr_v.at[0]],
                                ssems[q]).wait()

      gs = []
      for q in range(nbuf):
        gs.append(pltpu.async_copy(
            tab_hbm.at[c].at[si_v.at[pl.ds((blk + q) * _B, _B)]],
            bufs[q], gsems[q]))

      for q in range(nbuf):
        compute_w(blk + q)
        gs[q].wait()
        scale(bufs[q])
        pltpu.async_copy(bufs[q], acc_sh.at[dr_v.at[blk + q]],
                         ssems[q], add=True)

    for q in range(nbuf):
      pltpu.make_async_copy(bufs[q], acc_sh.at[dr_v.at[0]], ssems[q]).wait()

    plsc.subcore_barrier()

    @pl.loop(0, (_NCH + _NSUB - 1) // _NSUB)
    def _(t):
      idx = t * _NSUB + s

      @pl.when(idx < _NCH)
      def _():
        r0 = idx * _B
        pltpu.sync_copy(acc_sh.at[pl.ds(r0, _B)], out_hbm.at[c, pl.ds(r0, _B)])

  return k(tabs, src, dst, a_s, a_d, m16)


def kernel(features, edges, W1, att_src1, att_dst1, b1,
           W2, att_src2, att_dst2, b2):
  src = edges[0]
  dst = edges[1]
  tabs1, a1s, a1d, m1 = _dense1(features, W1, att_src1, att_dst1)
  big_m1 = jnp.maximum(m1[0, 0] + m1[0, 1], 0.0)
  m16_1 = jnp.full((16,), big_m1, jnp.float32)
  part1 = _sc_layer(tabs1, src, dst, a1s.reshape(-1), a1d.reshape(-1),
                    m16_1, _D1, 2)

  tabs2, a2s, a2d, m2 = _mid(part1, b1.reshape(1, _H), W2, att_src2, att_dst2)
  big_m2 = jnp.maximum(m2[0, 0] + m2[0, 1], 0.0)
  m16_2 = jnp.full((16,), big_m2, jnp.float32)
  part2 = _sc_layer(tabs2, src, dst, a2s.reshape(-1), a2d.reshape(-1),
                    m16_2, _D2, 5)

  return _final(part2, b2.reshape(1, _C))


# speedup vs baseline: 1.0462x; 1.0462x over previous
import dataclasses
import functools

import jax
import jax.numpy as jnp
from jax import lax
from jax.experimental import pallas as pl
from jax.experimental.pallas import tpu as pltpu
from jax.experimental.pallas import tpu_sc as plsc

_N = 10000
_E = 320000
_F = 128
_H = 128
_C = 16

_D1 = 80
_D2 = 16

_NCORES = 2
_NSUB = 16
_CH = _E // _NSUB
_B = 80
_NB = _CH // _B
_NCH = _N // _B
_DT = 4000
_SEG = _CH // _DT


def _dense1(features, w1, att_s, att_d):
  def body(x_ref, w_ref, s_ref, d_ref, tab_ref, as_ref, ad_ref, m_ref):
    xw = jnp.dot(x_ref[...], w_ref[...], preferred_element_type=jnp.float32)
    hh = _H // 2
    ones_col = (lax.broadcasted_iota(jnp.int32, (_N, _D1 - hh), 1) == 0)
    ones_col = ones_col.astype(jnp.float32)
    tab_ref[0, :, :hh] = xw[:, :hh]
    tab_ref[0, :, hh:] = ones_col
    tab_ref[1, :, :hh] = xw[:, hh:]
    tab_ref[1, :, hh:] = ones_col
    a_s = jnp.sum(xw * s_ref[...], axis=1)
    a_d = jnp.sum(xw * d_ref[...], axis=1)
    as_ref[...] = a_s[None, :]
    ad_ref[...] = a_d[None, :]
    m_ref[...] = jnp.concatenate(
        [jnp.max(a_s)[None, None], jnp.max(a_d)[None, None]], axis=1)

  return pl.pallas_call(
      body,
      out_shape=[
          jax.ShapeDtypeStruct((_NCORES, _N, _D1), jnp.float32),
          jax.ShapeDtypeStruct((1, _N), jnp.float32),
          jax.ShapeDtypeStruct((1, _N), jnp.float32),
          jax.ShapeDtypeStruct((1, 2), jnp.float32),
      ],
  )(features, w1, att_s, att_d)


def _mid(partial1, b1, w2, att_s, att_d):
  def body(p_ref, b_ref, w_ref, s_ref, d_ref, tab_ref, as_ref, ad_ref, m_ref):
    hh = _H // 2
    num = jnp.concatenate(
        [p_ref[0, :_N, :hh], p_ref[1, :_N, :hh]], axis=1)
    den = p_ref[0, :_N, hh:hh + 1] + 1e-16
    h = jax.nn.relu(num / den + b_ref[...])
    xw = jnp.dot(h, w_ref[...], preferred_element_type=jnp.float32)
    ch = _C // 2
    ones_col = (lax.broadcasted_iota(jnp.int32, (_N, _D2 - ch), 1) == 0)
    ones_col = ones_col.astype(jnp.float32)
    tab_ref[0, :, :ch] = xw[:, :ch]
    tab_ref[0, :, ch:] = ones_col
    tab_ref[1, :, :ch] = xw[:, ch:]
    tab_ref[1, :, ch:] = ones_col
    a_s = jnp.sum(xw * s_ref[...], axis=1)
    a_d = jnp.sum(xw * d_ref[...], axis=1)
    as_ref[...] = a_s[None, :]
    ad_ref[...] = a_d[None, :]
    m_ref[...] = jnp.concatenate(
        [jnp.max(a_s)[None, None], jnp.max(a_d)[None, None]], axis=1)

  return pl.pallas_call(
      body,
      out_shape=[
          jax.ShapeDtypeStruct((_NCORES, _N, _D2), jnp.float32),
          jax.ShapeDtypeStruct((1, _N), jnp.float32),
          jax.ShapeDtypeStruct((1, _N), jnp.float32),
          jax.ShapeDtypeStruct((1, 2), jnp.float32),
      ],
  )(partial1, b1, w2, att_s, att_d)


def _final(partial2, b2):
  def body(p_ref, b_ref, o_ref):
    ch = _C // 2
    num = jnp.concatenate(
        [p_ref[0, :_N, :ch], p_ref[1, :_N, :ch]], axis=1)
    den = p_ref[0, :_N, ch:ch + 1] + 1e-16
    o = num / den + b_ref[...]
    mx = jnp.max(o, axis=1, keepdims=True)
    shifted = o - mx
    lse = jnp.log(jnp.sum(jnp.exp(shifted), axis=1, keepdims=True))
    o_ref[...] = shifted - lse

  return pl.pallas_call(
      body,
      out_shape=jax.ShapeDtypeStruct((_N, _C), jnp.float32),
  )(partial2, b2)


def _sc_layer(tabs, src, dst, a_s, a_d, m16, d, nbuf):
  nd = d // 16
  assert _NB % nbuf == 0
  mesh = plsc.VectorSubcoreMesh(core_axis_name="c", subcore_axis_name="s")
  cp = pltpu.CompilerParams()
  if "needs_layout_passes" in pltpu.CompilerParams.__dataclass_fields__:
    cp = dataclasses.replace(cp, needs_layout_passes=False)
  if "use_tc_tiling_on_sc" in pltpu.CompilerParams.__dataclass_fields__:
    cp = dataclasses.replace(cp, use_tc_tiling_on_sc=False)

  @functools.partial(
      pl.kernel,
      mesh=mesh,
      compiler_params=cp,
      out_type=jax.ShapeDtypeStruct((_NCORES, _N, d), jnp.float32),
      scratch_types=[
          pltpu.VMEM((_N,), jnp.float32),
          pltpu.VMEM((_N,), jnp.float32),
          pltpu.VMEM((16,), jnp.float32),
          pltpu.VMEM((_CH,), jnp.int32),
          pltpu.VMEM((_NB, _B), jnp.int32),
          pltpu.VMEM((_DT,), jnp.int32),
          pltpu.VMEM((_B,), jnp.float32),
      ] + [pltpu.VMEM((_B, d), jnp.float32) for _ in range(nbuf)]
      + [pltpu.VMEM_SHARED((_N, d), jnp.float32)]
      + [pltpu.SemaphoreType.DMA for _ in range(2 * nbuf)],
  )
  def k(tab_hbm, src_hbm, dst_hbm, as_hbm, ad_hbm, m_hbm, out_hbm,
        as_v, ad_v, m_v, si_v, dr_v, dt_v, w_v, *rest):
    bufs = rest[:nbuf]
    acc_sh = rest[nbuf]
    gsems = rest[nbuf + 1:2 * nbuf + 1]
    ssems = rest[2 * nbuf + 1:]
    rows_a = bufs[0]
    c = lax.axis_index("c")
    s = lax.axis_index("s")

    pltpu.sync_copy(as_hbm, as_v)
    pltpu.sync_copy(ad_hbm, ad_v)
    pltpu.sync_copy(m_hbm, m_v)
    pltpu.sync_copy(src_hbm.at[pl.ds(s * _CH, _CH)], si_v)

    for seg in range(_SEG):
      pltpu.sync_copy(dst_hbm.at[pl.ds(s * _CH + seg * _DT, _DT)], dt_v)

      @pl.loop(0, _DT // _B)
      def _(b):
        for jj in range(_B // 16):
          dr_v[seg * (_DT // _B) + b, pl.ds(jj * 16, 16)] = (
              dt_v[pl.ds(b * _B + jj * 16, 16)])

    @pl.loop(0, _B)
    def _(r):
      for ch in range(nd):
        rows_a[r, pl.ds(ch * 16, 16)] = jnp.zeros((16,), jnp.float32)

    @pl.loop(0, (_NCH + _NSUB - 1) // _NSUB)
    def _(t):
      idx = t * _NSUB + s

      @pl.when(idx < _NCH)
      def _():
        pltpu.sync_copy(rows_a, acc_sh.at[pl.ds(idx * _B, _B)])

    plsc.subcore_barrier()

    mvec = m_v[...]

    def compute_w(blk):
      @plsc.parallel_loop(0, _B, 16, unroll=_B // 16)
      def _(j):
        sidx = si_v[pl.ds(blk * _B + j, 16)]
        didx = dr_v[blk, pl.ds(j, 16)]
        e = plsc.load_gather(as_v, [sidx]) + plsc.load_gather(ad_v, [didx])
        e = jnp.maximum(e, 0.2 * e)
        w_v[pl.ds(j, 16)] = jnp.exp(e - mvec)

    zc = jnp.zeros((16,), jnp.int32)

    def scale(rows_v):
      @plsc.parallel_loop(0, _B, unroll=8)
      def _(i):
        ws = plsc.load_gather(w_v, [zc + i])
        for ch in range(nd):
          sl = pl.ds(ch * 16, 16)
          rows_v[i, sl] = rows_v[i, sl] * ws

    @pl.loop(0, _NB, step=nbuf)
    def _(blk):
      @pl.when(blk > 0)
      def _():
        for q in range(nbuf):
          pltpu.make_async_copy(bufs[q], acc_sh.at[dr_v.at[0]],
                                ssems[q]).wait()

      gs = []
      for q in range(nbuf):
        gs.append(pltpu.async_copy(
            tab_hbm.at[c].at[si_v.at[pl.ds((blk + q) * _B, _B)]],
            bufs[q], gsems[q]))

      for q in range(nbuf):
        compute_w(blk + q)
        gs[q].wait()
        scale(bufs[q])
        pltpu.async_copy(bufs[q], acc_sh.at[dr_v.at[blk + q]],
                         ssems[q], add=True)

    for q in range(nbuf):
      pltpu.make_async_copy(bufs[q], acc_sh.at[dr_v.at[0]], ssems[q]).wait()

    plsc.subcore_barrier()

    @pl.loop(0, (_NCH + _NSUB - 1) // _NSUB)
    def _(t):
      idx = t * _NSUB + s

      @pl.when(idx < _NCH)
      def _():
        r0 = idx * _B
        pltpu.sync_copy(acc_sh.at[pl.ds(r0, _B)], out_hbm.at[c, pl.ds(r0, _B)])

  return k(tabs, src, dst, a_s, a_d, m16)


def kernel(features, edges, W1, att_src1, att_dst1, b1,
           W2, att_src2, att_dst2, b2):
  src = edges[0]
  dst = edges[1]
  tabs1, a1s, a1d, m1 = _dense1(features, W1, att_src1, att_dst1)
  big_m1 = jnp.maximum(m1[0, 0] + m1[0, 1], 0.0)
  m16_1 = jnp.full((16,), big_m1, jnp.float32)
  part1 = _sc_layer(tabs1, src, dst, a1s.reshape(-1), a1d.reshape(-1),
                    m16_1, _D1, 2)

  tabs2, a2s, a2d, m2 = _mid(part1, b1.reshape(1, _H), W2, att_src2, att_dst2)
  big_m2 = jnp.maximum(m2[0, 0] + m2[0, 1], 0.0)
  m16_2 = jnp.full((16,), big_m2, jnp.float32)
  part2 = _sc_layer(tabs2, src, dst, a2s.reshape(-1), a2d.reshape(-1),
                    m16_2, _D2, 10)

  return _final(part2, b2.reshape(1, _C))


# speedup vs baseline: 1.0717x; 1.0244x over previous
import dataclasses
import functools

import jax
import jax.numpy as jnp
from jax import lax
from jax.experimental import pallas as pl
from jax.experimental.pallas import tpu as pltpu
from jax.experimental.pallas import tpu_sc as plsc

_N = 10000
_E = 320000
_F = 128
_H = 128
_C = 16

_D1 = 80
_D2 = 32

_NCORES = 2
_NSUB = 16
_B = 80
_NCH = _N // _B
_DT = 2000


def _dense1(features, w1, att_s, att_d):
  def body(x_ref, w_ref, s_ref, d_ref, tab_ref, as_ref, ad_ref, m_ref):
    xw = jnp.dot(x_ref[...], w_ref[...], preferred_element_type=jnp.float32)
    hh = _H // 2
    ones_col = (lax.broadcasted_iota(jnp.int32, (_N, _D1 - hh), 1) == 0)
    ones_col = ones_col.astype(jnp.float32)
    tab_ref[0, :, :hh] = xw[:, :hh]
    tab_ref[0, :, hh:] = ones_col
    tab_ref[1, :, :hh] = xw[:, hh:]
    tab_ref[1, :, hh:] = ones_col
    a_s = jnp.sum(xw * s_ref[...], axis=1)
    a_d = jnp.sum(xw * d_ref[...], axis=1)
    as_ref[...] = a_s[None, :]
    ad_ref[...] = a_d[None, :]
    m_ref[...] = jnp.concatenate(
        [jnp.max(a_s)[None, None], jnp.max(a_d)[None, None]], axis=1)

  return pl.pallas_call(
      body,
      out_shape=[
          jax.ShapeDtypeStruct((_NCORES, _N, _D1), jnp.float32),
          jax.ShapeDtypeStruct((1, _N), jnp.float32),
          jax.ShapeDtypeStruct((1, _N), jnp.float32),
          jax.ShapeDtypeStruct((1, 2), jnp.float32),
      ],
  )(features, w1, att_s, att_d)


def _mid(partial1, b1, w2, att_s, att_d):
  def body(p_ref, b_ref, w_ref, s_ref, d_ref, tab_ref, as_ref, ad_ref, m_ref):
    hh = _H // 2
    num = jnp.concatenate(
        [p_ref[0, :_N, :hh], p_ref[1, :_N, :hh]], axis=1)
    den = p_ref[0, :_N, hh:hh + 1] + 1e-16
    h = jax.nn.relu(num / den + b_ref[...])
    xw = jnp.dot(h, w_ref[...], preferred_element_type=jnp.float32)
    ones_col = (lax.broadcasted_iota(jnp.int32, (_N, _D2 - _C), 1) == 0)
    ones_col = ones_col.astype(jnp.float32)
    tab_ref[:, :_C] = xw
    tab_ref[:, _C:] = ones_col
    a_s = jnp.sum(xw * s_ref[...], axis=1)
    a_d = jnp.sum(xw * d_ref[...], axis=1)
    as_ref[...] = a_s[None, :]
    ad_ref[...] = a_d[None, :]
    m_ref[...] = jnp.concatenate(
        [jnp.max(a_s)[None, None], jnp.max(a_d)[None, None]], axis=1)

  return pl.pallas_call(
      body,
      out_shape=[
          jax.ShapeDtypeStruct((_N, _D2), jnp.float32),
          jax.ShapeDtypeStruct((1, _N), jnp.float32),
          jax.ShapeDtypeStruct((1, _N), jnp.float32),
          jax.ShapeDtypeStruct((1, 2), jnp.float32),
      ],
  )(partial1, b1, w2, att_s, att_d)


def _final(partial2, b2):
  def body(p_ref, b_ref, o_ref):
    p = p_ref[0] + p_ref[1]
    num = p[:, :_C]
    den = p[:, _C:_C + 1] + 1e-16
    o = num / den + b_ref[...]
    mx = jnp.max(o, axis=1, keepdims=True)
    shifted = o - mx
    lse = jnp.log(jnp.sum(jnp.exp(shifted), axis=1, keepdims=True))
    o_ref[...] = shifted - lse

  return pl.pallas_call(
      body,
      out_shape=jax.ShapeDtypeStruct((_N, _C), jnp.float32),
  )(partial2, b2)


def _sc_layer(tabs, src, dst, a_s, a_d, m16, d, nbuf, edge_split=False):
  nd = d // 16
  ch_e = _E // (_NCORES * _NSUB) if edge_split else _E // _NSUB
  nb = ch_e // _B
  seg = ch_e // _DT
  assert nb % nbuf == 0 and ch_e % _DT == 0 and _DT % _B == 0
  mesh = plsc.VectorSubcoreMesh(core_axis_name="c", subcore_axis_name="s")
  cp = pltpu.CompilerParams()
  if "needs_layout_passes" in pltpu.CompilerParams.__dataclass_fields__:
    cp = dataclasses.replace(cp, needs_layout_passes=False)
  if "use_tc_tiling_on_sc" in pltpu.CompilerParams.__dataclass_fields__:
    cp = dataclasses.replace(cp, use_tc_tiling_on_sc=False)

  @functools.partial(
      pl.kernel,
      mesh=mesh,
      compiler_params=cp,
      out_type=jax.ShapeDtypeStruct((_NCORES, _N, d), jnp.float32),
      scratch_types=[
          pltpu.VMEM((_N,), jnp.float32),
          pltpu.VMEM((_N,), jnp.float32),
          pltpu.VMEM((16,), jnp.float32),
          pltpu.VMEM((ch_e,), jnp.int32),
          pltpu.VMEM((nb, _B), jnp.int32),
          pltpu.VMEM((_DT,), jnp.int32),
          pltpu.VMEM((_B,), jnp.float32),
      ] + [pltpu.VMEM((_B, d), jnp.float32) for _ in range(nbuf)]
      + [pltpu.VMEM_SHARED((_N, d), jnp.float32)]
      + [pltpu.SemaphoreType.DMA for _ in range(2 * nbuf)],
  )
  def k(tab_hbm, src_hbm, dst_hbm, as_hbm, ad_hbm, m_hbm, out_hbm,
        as_v, ad_v, m_v, si_v, dr_v, dt_v, w_v, *rest):
    bufs = rest[:nbuf]
    acc_sh = rest[nbuf]
    gsems = rest[nbuf + 1:2 * nbuf + 1]
    ssems = rest[2 * nbuf + 1:]
    rows_a = bufs[0]
    c = lax.axis_index("c")
    s = lax.axis_index("s")

    pltpu.sync_copy(as_hbm, as_v)
    pltpu.sync_copy(ad_hbm, ad_v)
    pltpu.sync_copy(m_hbm, m_v)
    if edge_split:
      ebase = c * (_E // _NCORES) + s * ch_e
    else:
      ebase = s * ch_e
    pltpu.sync_copy(src_hbm.at[pl.ds(ebase, ch_e)], si_v)

    for sg in range(seg):
      pltpu.sync_copy(dst_hbm.at[pl.ds(ebase + sg * _DT, _DT)], dt_v)

      @pl.loop(0, _DT // _B)
      def _(b):
        for jj in range(_B // 16):
          dr_v[sg * (_DT // _B) + b, pl.ds(jj * 16, 16)] = (
              dt_v[pl.ds(b * _B + jj * 16, 16)])

    @pl.loop(0, _B)
    def _(r):
      for ch in range(nd):
        rows_a[r, pl.ds(ch * 16, 16)] = jnp.zeros((16,), jnp.float32)

    @pl.loop(0, (_NCH + _NSUB - 1) // _NSUB)
    def _(t):
      idx = t * _NSUB + s

      @pl.when(idx < _NCH)
      def _():
        pltpu.sync_copy(rows_a, acc_sh.at[pl.ds(idx * _B, _B)])

    plsc.subcore_barrier()

    mvec = m_v[...]

    def compute_w(blk):
      @plsc.parallel_loop(0, _B, 16, unroll=_B // 16)
      def _(j):
        sidx = si_v[pl.ds(blk * _B + j, 16)]
        didx = dr_v[blk, pl.ds(j, 16)]
        e = plsc.load_gather(as_v, [sidx]) + plsc.load_gather(ad_v, [didx])
        e = jnp.maximum(e, 0.2 * e)
        w_v[pl.ds(j, 16)] = jnp.exp(e - mvec)

    zc = jnp.zeros((16,), jnp.int32)

    def scale(rows_v):
      @plsc.parallel_loop(0, _B, unroll=8)
      def _(i):
        ws = plsc.load_gather(w_v, [zc + i])
        for ch in range(nd):
          sl = pl.ds(ch * 16, 16)
          rows_v[i, sl] = rows_v[i, sl] * ws

    @pl.loop(0, nb, step=nbuf)
    def _(blk):
      @pl.when(blk > 0)
      def _():
        for q in range(nbuf):
          pltpu.make_async_copy(bufs[q], acc_sh.at[dr_v.at[0]],
                                ssems[q]).wait()

      my_tab = tab_hbm if edge_split else tab_hbm.at[c]
      gs = []
      for q in range(nbuf):
        gs.append(pltpu.async_copy(
            my_tab.at[si_v.at[pl.ds((blk + q) * _B, _B)]],
            bufs[q], gsems[q]))

      for q in range(nbuf):
        compute_w(blk + q)
        gs[q].wait()
        scale(bufs[q])
        pltpu.async_copy(bufs[q], acc_sh.at[dr_v.at[blk + q]],
                         ssems[q], add=True)

    for q in range(nbuf):
      pltpu.make_async_copy(bufs[q], acc_sh.at[dr_v.at[0]], ssems[q]).wait()

    plsc.subcore_barrier()

    @pl.loop(0, (_NCH + _NSUB - 1) // _NSUB)
    def _(t):
      idx = t * _NSUB + s

      @pl.when(idx < _NCH)
      def _():
        r0 = idx * _B
        pltpu.sync_copy(acc_sh.at[pl.ds(r0, _B)], out_hbm.at[c, pl.ds(r0, _B)])

  return k(tabs, src, dst, a_s, a_d, m16)


def kernel(features, edges, W1, att_src1, att_dst1, b1,
           W2, att_src2, att_dst2, b2):
  src = edges[0]
  dst = edges[1]
  tabs1, a1s, a1d, m1 = _dense1(features, W1, att_src1, att_dst1)
  big_m1 = jnp.maximum(m1[0, 0] + m1[0, 1], 0.0)
  m16_1 = jnp.full((16,), big_m1, jnp.float32)
  part1 = _sc_layer(tabs1, src, dst, a1s.reshape(-1), a1d.reshape(-1),
                    m16_1, _D1, 2)

  tabs2, a2s, a2d, m2 = _mid(part1, b1.reshape(1, _H), W2, att_src2, att_dst2)
  big_m2 = jnp.maximum(m2[0, 0] + m2[0, 1], 0.0)
  m16_2 = jnp.full((16,), big_m2, jnp.float32)
  part2 = _sc_layer(tabs2, src, dst, a2s.reshape(-1), a2d.reshape(-1),
                    m16_2, _D2, 5, edge_split=True)

  return _final(part2, b2.reshape(1, _C))


# speedup vs baseline: 1.0965x; 1.0232x over previous
import dataclasses
import functools

import jax
import jax.numpy as jnp
from jax import lax
from jax.experimental import pallas as pl
from jax.experimental.pallas import tpu as pltpu
from jax.experimental.pallas import tpu_sc as plsc

_N = 10000
_E = 320000
_F = 128
_H = 128
_C = 16

_D1 = 80
_D2 = 32

_NCORES = 2
_NSUB = 16
_B = 80
_NCH = _N // _B
_DT = 2000


def _dense1(features, w1, att_s, att_d):
  def body(x_ref, w_ref, s_ref, d_ref, tab_ref, as_ref, ad_ref, m_ref):
    xw = jnp.dot(x_ref[...], w_ref[...], preferred_element_type=jnp.float32)
    hh = _H // 2
    ones_col = (lax.broadcasted_iota(jnp.int32, (_N, _D1 - hh), 1) == 0)
    ones_col = ones_col.astype(jnp.float32)
    tab_ref[0, :, :hh] = xw[:, :hh]
    tab_ref[0, :, hh:] = ones_col
    tab_ref[1, :, :hh] = xw[:, hh:]
    tab_ref[1, :, hh:] = ones_col
    a_s = jnp.sum(xw * s_ref[...], axis=1)
    a_d = jnp.sum(xw * d_ref[...], axis=1)
    as_ref[...] = a_s[None, :]
    ad_ref[...] = a_d[None, :]
    m_ref[...] = jnp.concatenate(
        [jnp.max(a_s)[None, None], jnp.max(a_d)[None, None]], axis=1)

  return pl.pallas_call(
      body,
      out_shape=[
          jax.ShapeDtypeStruct((_NCORES, _N, _D1), jnp.float32),
          jax.ShapeDtypeStruct((1, _N), jnp.float32),
          jax.ShapeDtypeStruct((1, _N), jnp.float32),
          jax.ShapeDtypeStruct((1, 2), jnp.float32),
      ],
  )(features, w1, att_s, att_d)


def _mid(partial1, b1, w2, att_s, att_d):
  def body(p_ref, b_ref, w_ref, s_ref, d_ref, tab_ref, as_ref, ad_ref, m_ref):
    hh = _H // 2
    num = jnp.concatenate(
        [p_ref[0, :_N, :hh], p_ref[1, :_N, :hh]], axis=1)
    den = p_ref[0, :_N, hh:hh + 1] + 1e-16
    h = jax.nn.relu(num / den + b_ref[...])
    xw = jnp.dot(h, w_ref[...], preferred_element_type=jnp.float32)
    ones_col = (lax.broadcasted_iota(jnp.int32, (_N, _D2 - _C), 1) == 0)
    ones_col = ones_col.astype(jnp.float32)
    tab_ref[:, :_C] = xw
    tab_ref[:, _C:] = ones_col
    a_s = jnp.sum(xw * s_ref[...], axis=1)
    a_d = jnp.sum(xw * d_ref[...], axis=1)
    as_ref[...] = a_s[None, :]
    ad_ref[...] = a_d[None, :]
    m_ref[...] = jnp.concatenate(
        [jnp.max(a_s)[None, None], jnp.max(a_d)[None, None]], axis=1)

  return pl.pallas_call(
      body,
      out_shape=[
          jax.ShapeDtypeStruct((_N, _D2), jnp.float32),
          jax.ShapeDtypeStruct((1, _N), jnp.float32),
          jax.ShapeDtypeStruct((1, _N), jnp.float32),
          jax.ShapeDtypeStruct((1, 2), jnp.float32),
      ],
  )(partial1, b1, w2, att_s, att_d)


def _final(partial2, b2):
  def body(p_ref, b_ref, o_ref):
    p = p_ref[0] + p_ref[1]
    num = p[:, :_C]
    den = p[:, _C:_C + 1] + 1e-16
    o = num / den + b_ref[...]
    mx = jnp.max(o, axis=1, keepdims=True)
    shifted = o - mx
    lse = jnp.log(jnp.sum(jnp.exp(shifted), axis=1, keepdims=True))
    o_ref[...] = shifted - lse

  return pl.pallas_call(
      body,
      out_shape=jax.ShapeDtypeStruct((_N, _C), jnp.float32),
  )(partial2, b2)


def _sc_layer(tabs, src, dst, a_s, a_d, m16, d, nbuf, edge_split=False):
  nd = d // 16
  ch_e = _E // (_NCORES * _NSUB) if edge_split else _E // _NSUB
  nb = ch_e // _B
  seg = ch_e // _DT
  assert nb % nbuf == 0 and ch_e % _DT == 0 and _DT % _B == 0
  mesh = plsc.VectorSubcoreMesh(core_axis_name="c", subcore_axis_name="s")
  cp = pltpu.CompilerParams()
  if "needs_layout_passes" in pltpu.CompilerParams.__dataclass_fields__:
    cp = dataclasses.replace(cp, needs_layout_passes=False)
  if "use_tc_tiling_on_sc" in pltpu.CompilerParams.__dataclass_fields__:
    cp = dataclasses.replace(cp, use_tc_tiling_on_sc=False)

  @functools.partial(
      pl.kernel,
      mesh=mesh,
      compiler_params=cp,
      out_type=jax.ShapeDtypeStruct((_NCORES, _N, d), jnp.float32),
      scratch_types=[
          pltpu.VMEM((_N,), jnp.float32),
          pltpu.VMEM((_N,), jnp.float32),
          pltpu.VMEM((16,), jnp.float32),
          pltpu.VMEM((ch_e,), jnp.int32),
          pltpu.VMEM((nb, _B), jnp.int32),
          pltpu.VMEM((_DT,), jnp.int32),
          pltpu.VMEM((_B,), jnp.float32),
      ] + [pltpu.VMEM((_B, d), jnp.float32) for _ in range(nbuf)]
      + [pltpu.VMEM_SHARED((_N, d), jnp.float32)]
      + [pltpu.SemaphoreType.DMA for _ in range(2 * nbuf)],
  )
  def k(tab_hbm, src_hbm, dst_hbm, as_hbm, ad_hbm, m_hbm, out_hbm,
        as_v, ad_v, m_v, si_v, dr_v, dt_v, w_v, *rest):
    bufs = rest[:nbuf]
    acc_sh = rest[nbuf]
    gsems = rest[nbuf + 1:2 * nbuf + 1]
    ssems = rest[2 * nbuf + 1:]
    rows_a = bufs[0]
    c = lax.axis_index("c")
    s = lax.axis_index("s")

    if edge_split:
      ebase = c * (_E // _NCORES) + s * ch_e
    else:
      ebase = s * ch_e
    setup_cps = [
        pltpu.async_copy(as_hbm, as_v, gsems[0]),
        pltpu.async_copy(ad_hbm, ad_v, gsems[1]),
        pltpu.async_copy(m_hbm, m_v, ssems[0]),
        pltpu.async_copy(src_hbm.at[pl.ds(ebase, ch_e)], si_v, ssems[1]),
    ]

    for sg in range(seg):
      pltpu.sync_copy(dst_hbm.at[pl.ds(ebase + sg * _DT, _DT)], dt_v)

      @pl.loop(0, _DT // _B)
      def _(b):
        for jj in range(_B // 16):
          dr_v[sg * (_DT // _B) + b, pl.ds(jj * 16, 16)] = (
              dt_v[pl.ds(b * _B + jj * 16, 16)])

    @pl.loop(0, _B)
    def _(r):
      for ch in range(nd):
        rows_a[r, pl.ds(ch * 16, 16)] = jnp.zeros((16,), jnp.float32)

    nz = (_NCH + _NSUB - 1) // _NSUB

    @pl.loop(0, nz)
    def _(t):
      idx = t * _NSUB + s

      @pl.when(idx < _NCH)
      def _():
        pltpu.async_copy(rows_a, acc_sh.at[pl.ds(idx * _B, _B)], gsems[0])

    for cp in setup_cps:
      cp.wait()

    @pl.loop(0, nz)
    def _(t):
      idx = t * _NSUB + s

      @pl.when(idx < _NCH)
      def _():
        pltpu.make_async_copy(rows_a, acc_sh.at[pl.ds(0, _B)],
                              gsems[0]).wait()

    plsc.subcore_barrier()

    mvec = m_v[...]

    def compute_w(blk):
      @plsc.parallel_loop(0, _B, 16, unroll=_B // 16)
      def _(j):
        sidx = si_v[pl.ds(blk * _B + j, 16)]
        didx = dr_v[blk, pl.ds(j, 16)]
        e = plsc.load_gather(as_v, [sidx]) + plsc.load_gather(ad_v, [didx])
        e = jnp.maximum(e, 0.2 * e)
        w_v[pl.ds(j, 16)] = jnp.exp(e - mvec)

    zc = jnp.zeros((16,), jnp.int32)

    def scale(rows_v):
      @plsc.parallel_loop(0, _B, unroll=8)
      def _(i):
        ws = plsc.load_gather(w_v, [zc + i])
        for ch in range(nd):
          sl = pl.ds(ch * 16, 16)
          rows_v[i, sl] = rows_v[i, sl] * ws

    @pl.loop(0, nb, step=nbuf)
    def _(blk):
      @pl.when(blk > 0)
      def _():
        for q in range(nbuf):
          pltpu.make_async_copy(bufs[q], acc_sh.at[dr_v.at[0]],
                                ssems[q]).wait()

      my_tab = tab_hbm if edge_split else tab_hbm.at[c]
      gs = []
      for q in range(nbuf):
        gs.append(pltpu.async_copy(
            my_tab.at[si_v.at[pl.ds((blk + q) * _B, _B)]],
            bufs[q], gsems[q]))

      for q in range(nbuf):
        compute_w(blk + q)
        gs[q].wait()
        scale(bufs[q])
        pltpu.async_copy(bufs[q], acc_sh.at[dr_v.at[blk + q]],
                         ssems[q], add=True)

    for q in range(nbuf):
      pltpu.make_async_copy(bufs[q], acc_sh.at[dr_v.at[0]], ssems[q]).wait()

    plsc.subcore_barrier()

    @pl.loop(0, nz)
    def _(t):
      idx = t * _NSUB + s

      @pl.when(idx < _NCH)
      def _():
        r0 = idx * _B
        pltpu.async_copy(acc_sh.at[pl.ds(r0, _B)], out_hbm.at[c, pl.ds(r0, _B)],
                         gsems[0])

    @pl.loop(0, nz)
    def _(t):
      idx = t * _NSUB + s

      @pl.when(idx < _NCH)
      def _():
        pltpu.make_async_copy(acc_sh.at[pl.ds(0, _B)],
                              out_hbm.at[c, pl.ds(0, _B)], gsems[0]).wait()

  return k(tabs, src, dst, a_s, a_d, m16)


def kernel(features, edges, W1, att_src1, att_dst1, b1,
           W2, att_src2, att_dst2, b2):
  src = edges[0]
  dst = edges[1]
  tabs1, a1s, a1d, m1 = _dense1(features, W1, att_src1, att_dst1)
  big_m1 = jnp.maximum(m1[0, 0] + m1[0, 1], 0.0)
  m16_1 = jnp.full((16,), big_m1, jnp.float32)
  part1 = _sc_layer(tabs1, src, dst, a1s.reshape(-1), a1d.reshape(-1),
                    m16_1, _D1, 2)

  tabs2, a2s, a2d, m2 = _mid(part1, b1.reshape(1, _H), W2, att_src2, att_dst2)
  big_m2 = jnp.maximum(m2[0, 0] + m2[0, 1], 0.0)
  m16_2 = jnp.full((16,), big_m2, jnp.float32)
  part2 = _sc_layer(tabs2, src, dst, a2s.reshape(-1), a2d.reshape(-1),
                    m16_2, _D2, 5, edge_split=True)

  return _final(part2, b2.reshape(1, _C))


# speedup vs baseline: 1.1416x; 1.0411x over previous
import dataclasses
import functools

import jax
import jax.numpy as jnp
from jax import lax
from jax.experimental import pallas as pl
from jax.experimental.pallas import tpu as pltpu
from jax.experimental.pallas import tpu_sc as plsc

_N = 10000
_E = 320000
_F = 128
_H = 128
_C = 16

_D1 = 64
_D2 = 32

_NCORES = 2
_NSUB = 16
_B = 80
_NCH = _N // _B
_DT = 2000


def _dense1(features, w1, att_s, att_d):
  def body(x_ref, w_ref, s_ref, d_ref, tab_ref, as_ref, ad_ref, m_ref):
    xw = jnp.dot(x_ref[...], w_ref[...], preferred_element_type=jnp.float32)
    hh = _H // 2
    tab_ref[0, :, :] = xw[:, :hh]
    tab_ref[1, :, :] = xw[:, hh:]
    a_s = jnp.sum(xw * s_ref[...], axis=1)
    a_d = jnp.sum(xw * d_ref[...], axis=1)
    as_ref[...] = a_s[None, :]
    ad_ref[...] = a_d[None, :]
    m_ref[...] = jnp.concatenate(
        [jnp.max(a_s)[None, None], jnp.max(a_d)[None, None]], axis=1)

  return pl.pallas_call(
      body,
      out_shape=[
          jax.ShapeDtypeStruct((_NCORES, _N, _D1), jnp.float32),
          jax.ShapeDtypeStruct((1, _N), jnp.float32),
          jax.ShapeDtypeStruct((1, _N), jnp.float32),
          jax.ShapeDtypeStruct((1, 2), jnp.float32),
      ],
  )(features, w1, att_s, att_d)


def _mid(partial1, partial1_d, b1, w2, att_s, att_d):
  def body(p_ref, pd_ref, b_ref, w_ref, s_ref, d_ref,
           tab_ref, as_ref, ad_ref, m_ref):
    num = jnp.concatenate([p_ref[0], p_ref[1]], axis=1)
    den = pd_ref[0, :, 0:1] + 1e-16
    h = jax.nn.relu(num / den + b_ref[...])
    xw = jnp.dot(h, w_ref[...], preferred_element_type=jnp.float32)
    ones_col = (lax.broadcasted_iota(jnp.int32, (_N, _D2 - _C), 1) == 0)
    ones_col = ones_col.astype(jnp.float32)
    tab_ref[:, :_C] = xw
    tab_ref[:, _C:] = ones_col
    a_s = jnp.sum(xw * s_ref[...], axis=1)
    a_d = jnp.sum(xw * d_ref[...], axis=1)
    as_ref[...] = a_s[None, :]
    ad_ref[...] = a_d[None, :]
    m_ref[...] = jnp.concatenate(
        [jnp.max(a_s)[None, None], jnp.max(a_d)[None, None]], axis=1)

  return pl.pallas_call(
      body,
      out_shape=[
          jax.ShapeDtypeStruct((_N, _D2), jnp.float32),
          jax.ShapeDtypeStruct((1, _N), jnp.float32),
          jax.ShapeDtypeStruct((1, _N), jnp.float32),
          jax.ShapeDtypeStruct((1, 2), jnp.float32),
      ],
  )(partial1, partial1_d, b1, w2, att_s, att_d)


def _final(partial2, b2):
  def body(p_ref, b_ref, o_ref):
    p = p_ref[0] + p_ref[1]
    num = p[:, :_C]
    den = p[:, _C:_C + 1] + 1e-16
    o = num / den + b_ref[...]
    mx = jnp.max(o, axis=1, keepdims=True)
    shifted = o - mx
    lse = jnp.log(jnp.sum(jnp.exp(shifted), axis=1, keepdims=True))
    o_ref[...] = shifted - lse

  return pl.pallas_call(
      body,
      out_shape=jax.ShapeDtypeStruct((_N, _C), jnp.float32),
  )(partial2, b2)


def _sc_layer(tabs, src, dst, a_s, a_d, m16, d, nbuf, edge_split=False,
              denom=False):
  nd = d // 16
  ch_e = _E // (_NCORES * _NSUB) if edge_split else _E // _NSUB
  nb = ch_e // _B
  seg = ch_e // _DT
  assert nb % nbuf == 0 and ch_e % _DT == 0 and _DT % _B == 0
  mesh = plsc.VectorSubcoreMesh(core_axis_name="c", subcore_axis_name="s")
  cp = pltpu.CompilerParams()
  if "needs_layout_passes" in pltpu.CompilerParams.__dataclass_fields__:
    cp = dataclasses.replace(cp, needs_layout_passes=False)
  if "use_tc_tiling_on_sc" in pltpu.CompilerParams.__dataclass_fields__:
    cp = dataclasses.replace(cp, use_tc_tiling_on_sc=False)

  ndb = nbuf if denom else 0
  if denom:
    outs = [jax.ShapeDtypeStruct((_NCORES, _N, d), jnp.float32),
            jax.ShapeDtypeStruct((_NCORES, _N, 16), jnp.float32)]
  else:
    outs = jax.ShapeDtypeStruct((_NCORES, _N, d), jnp.float32)

  @functools.partial(
      pl.kernel,
      mesh=mesh,
      compiler_params=cp,
      out_type=outs,
      scratch_types=[
          pltpu.VMEM((_N,), jnp.float32),
          pltpu.VMEM((_N,), jnp.float32),
          pltpu.VMEM((16,), jnp.float32),
          pltpu.VMEM((ch_e,), jnp.int32),
          pltpu.VMEM((nb, _B), jnp.int32),
          pltpu.VMEM((_DT,), jnp.int32),
          pltpu.VMEM((_B,), jnp.float32),
      ] + [pltpu.VMEM((_B, d), jnp.float32) for _ in range(nbuf)]
      + [pltpu.VMEM((_B, 16), jnp.float32) for _ in range(ndb)]
      + [pltpu.VMEM_SHARED((_N, d), jnp.float32)]
      + ([pltpu.VMEM_SHARED((_N, 16), jnp.float32)] if denom else [])
      + [pltpu.SemaphoreType.DMA for _ in range(2 * nbuf + ndb)],
  )
  def k(tab_hbm, src_hbm, dst_hbm, as_hbm, ad_hbm, m_hbm, *rest):
    pos = 2 if denom else 1
    out_hbm = rest[0]
    outd_hbm = rest[1] if denom else None
    as_v, ad_v, m_v, si_v, dr_v, dt_v, w_v = rest[pos:pos + 7]
    pos += 7
    bufs = rest[pos:pos + nbuf]
    pos += nbuf
    dbufs = rest[pos:pos + ndb]
    pos += ndb
    acc_sh = rest[pos]
    pos += 1
    if denom:
      accd_sh = rest[pos]
      pos += 1
    gsems = rest[pos:pos + nbuf]
    ssems = rest[pos + nbuf:pos + 2 * nbuf]
    dsems = rest[pos + 2 * nbuf:]
    rows_a = bufs[0]
    c = lax.axis_index("c")
    s = lax.axis_index("s")

    if edge_split:
      ebase = c * (_E // _NCORES) + s * ch_e
    else:
      ebase = s * ch_e
    setup_cps = [
        pltpu.async_copy(as_hbm, as_v, gsems[0]),
        pltpu.async_copy(ad_hbm, ad_v, gsems[1]),
        pltpu.async_copy(m_hbm, m_v, ssems[0]),
        pltpu.async_copy(src_hbm.at[pl.ds(ebase, ch_e)], si_v, ssems[1]),
    ]

    for sg in range(seg):
      pltpu.sync_copy(dst_hbm.at[pl.ds(ebase + sg * _DT, _DT)], dt_v)

      @pl.loop(0, _DT // _B)
      def _(b):
        for jj in range(_B // 16):
          dr_v[sg * (_DT // _B) + b, pl.ds(jj * 16, 16)] = (
              dt_v[pl.ds(b * _B + jj * 16, 16)])

    @pl.loop(0, _B)
    def _(r):
      for ch in range(nd):
        rows_a[r, pl.ds(ch * 16, 16)] = jnp.zeros((16,), jnp.float32)
      if denom:
        dbufs[0][r, pl.ds(0, 16)] = jnp.zeros((16,), jnp.float32)

    nz = (_NCH + _NSUB - 1) // _NSUB

    @pl.loop(0, nz)
    def _(t):
      idx = t * _NSUB + s

      @pl.when(idx < _NCH)
      def _():
        pltpu.async_copy(rows_a, acc_sh.at[pl.ds(idx * _B, _B)], gsems[0])
        if denom:
          pltpu.async_copy(dbufs[0], accd_sh.at[pl.ds(idx * _B, _B)],
                           gsems[1])

    for cp in setup_cps:
      cp.wait()

    @pl.loop(0, nz)
    def _(t):
      idx = t * _NSUB + s

      @pl.when(idx < _NCH)
      def _():
        pltpu.make_async_copy(rows_a, acc_sh.at[pl.ds(0, _B)],
                              gsems[0]).wait()
        if denom:
          pltpu.make_async_copy(dbufs[0], accd_sh.at[pl.ds(0, _B)],
                                gsems[1]).wait()

    plsc.subcore_barrier()

    mvec = m_v[...]

    def compute_w(blk):
      @plsc.parallel_loop(0, _B, 16, unroll=_B // 16)
      def _(j):
        sidx = si_v[pl.ds(blk * _B + j, 16)]
        didx = dr_v[blk, pl.ds(j, 16)]
        e = plsc.load_gather(as_v, [sidx]) + plsc.load_gather(ad_v, [didx])
        e = jnp.maximum(e, 0.2 * e)
        w_v[pl.ds(j, 16)] = jnp.exp(e - mvec)

    zc = jnp.zeros((16,), jnp.int32)
    if denom:
      lane = lax.iota(jnp.int32, 16)
      e0 = jnp.where(lane == 0, jnp.float32(1), jnp.float32(0))

    def scale(rows_v, db_v):
      @plsc.parallel_loop(0, _B, unroll=8)
      def _(i):
        ws = plsc.load_gather(w_v, [zc + i])
        for ch in range(nd):
          sl = pl.ds(ch * 16, 16)
          rows_v[i, sl] = rows_v[i, sl] * ws
        if denom:
          db_v[i, pl.ds(0, 16)] = ws * e0

    @pl.loop(0, nb, step=nbuf)
    def _(blk):
      @pl.when(blk > 0)
      def _():
        for q in range(nbuf):
          pltpu.make_async_copy(bufs[q], acc_sh.at[dr_v.at[0]],
                                ssems[q]).wait()
          if denom:
            pltpu.make_async_copy(dbufs[q], accd_sh.at[dr_v.at[0]],
                                  dsems[q]).wait()

      my_tab = tab_hbm if edge_split else tab_hbm.at[c]
      gs = []
      for q in range(nbuf):
        gs.append(pltpu.async_copy(
            my_tab.at[si_v.at[pl.ds((blk + q) * _B, _B)]],
            bufs[q], gsems[q]))

      for q in range(nbuf):
        compute_w(blk + q)
        gs[q].wait()
        scale(bufs[q], dbufs[q] if denom else None)
        pltpu.async_copy(bufs[q], acc_sh.at[dr_v.at[blk + q]],
                         ssems[q], add=True)
        if denom:
          pltpu.async_copy(dbufs[q], accd_sh.at[dr_v.at[blk + q]],
                           dsems[q], add=True)

    for q in range(nbuf):
      pltpu.make_async_copy(bufs[q], acc_sh.at[dr_v.at[0]], ssems[q]).wait()
      if denom:
        pltpu.make_async_copy(dbufs[q], accd_sh.at[dr_v.at[0]],
                              dsems[q]).wait()

    plsc.subcore_barrier()

    @pl.loop(0, nz)
    def _(t):
      idx = t * _NSUB + s

      @pl.when(idx < _NCH)
      def _():
        r0 = idx * _B
        pltpu.async_copy(acc_sh.at[pl.ds(r0, _B)], out_hbm.at[c, pl.ds(r0, _B)],
                         gsems[0])
        if denom:
          pltpu.async_copy(accd_sh.at[pl.ds(r0, _B)],
                           outd_hbm.at[c, pl.ds(r0, _B)], gsems[1])

    @pl.loop(0, nz)
    def _(t):
      idx = t * _NSUB + s

      @pl.when(idx < _NCH)
      def _():
        pltpu.make_async_copy(acc_sh.at[pl.ds(0, _B)],
                              out_hbm.at[c, pl.ds(0, _B)], gsems[0]).wait()
        if denom:
          pltpu.make_async_copy(accd_sh.at[pl.ds(0, _B)],
                                outd_hbm.at[c, pl.ds(0, _B)], gsems[1]).wait()

  return k(tabs, src, dst, a_s, a_d, m16)


def kernel(features, edges, W1, att_src1, att_dst1, b1,
           W2, att_src2, att_dst2, b2):
  src = edges[0]
  dst = edges[1]
  tabs1, a1s, a1d, m1 = _dense1(features, W1, att_src1, att_dst1)
  big_m1 = jnp.maximum(m1[0, 0] + m1[0, 1], 0.0)
  m16_1 = jnp.full((16,), big_m1, jnp.float32)
  part1, part1_d = _sc_layer(tabs1, src, dst, a1s.reshape(-1),
                             a1d.reshape(-1), m16_1, _D1, 2, denom=True)

  tabs2, a2s, a2d, m2 = _mid(part1, part1_d, b1.reshape(1, _H), W2,
                             att_src2, att_dst2)
  big_m2 = jnp.maximum(m2[0, 0] + m2[0, 1], 0.0)
  m16_2 = jnp.full((16,), big_m2, jnp.float32)
  part2 = _sc_layer(tabs2, src, dst, a2s.reshape(-1), a2d.reshape(-1),
                    m16_2, _D2, 5, edge_split=True)

  return _final(part2, b2.reshape(1, _C))
